# baseline (speedup 1.0000x reference)
import functools

import jax
import jax.numpy as jnp
from jax import lax
from jax.experimental import pallas as pl
from jax.experimental.pallas import tpu as pltpu
from jax.experimental.pallas import tpu_sc as plsc

_N = 10000
_E = 320000
_D = 128
_H = 128
_R = 8
_NB = 8
_C = 16

_NC, _NS, _L = 2, 16, 16
_NW = _NC * _NS
_EPW = _E // _NW
_CB = 64
_MR = 8
_WM = 4
_WCB = _CB * _WM
_MCB = _CB * _MR
_ROWS = _E // _CB
_NMAC = _ROWS // _MR
_BPR = _CB // _L
_SEGP = 81920
_SPS = _SEGP // _NS
_NP = 10240
_RPS = _NP // _NS
_ZR = 8

_HI = lax.Precision.HIGHEST


def _mesh():
    return plsc.VectorSubcoreMesh(
        core_axis_name="c", subcore_axis_name="s",
        num_cores=_NC, num_subcores=_NS)


def _sc_prep(src2, dst2, et2):
    @functools.partial(
        pl.kernel,
        out_type=(
            jax.ShapeDtypeStruct((_NC, _SEGP), jnp.float32),
            jax.ShapeDtypeStruct((_ROWS, _CB), jnp.int32),
            jax.ShapeDtypeStruct((_ROWS, _CB), jnp.int32),
        ),
        mesh=_mesh(),
        scratch_types=[
            pltpu.VMEM((_MR, _CB), jnp.int32),
            pltpu.VMEM((_MR, _CB), jnp.int32),
            pltpu.VMEM((_MR, _CB), jnp.int32),
            pltpu.VMEM((_MR, _CB), jnp.int32),
            pltpu.VMEM((_MR, _CB), jnp.int32),
            pltpu.VMEM((_CB,), jnp.float32),
            pltpu.VMEM((_SPS,), jnp.float32),
            pltpu.VMEM_SHARED((_SEGP,), jnp.float32),
            pltpu.SemaphoreType.DMA,
        ],
    )
    def body(src_h, dst_h, et_h, cnt_h, gidx_h, seg_h,
             src_v, dst_v, et_v, gi_v, sg_v, ones_v, z_v, acc, sem):
        c = lax.axis_index("c")
        s = lax.axis_index("s")
        wid = s * _NC + c
        nm = (_NMAC - wid + _NW - 1) // _NW

        def initz(i, _):
            z_v[pl.ds(i * _L, _L)] = jnp.zeros((_L,), jnp.float32)
            return 0
        lax.fori_loop(0, _SPS // _L, initz, 0)

        for j in range(_CB // _L):
            ones_v[pl.ds(j * _L, _L)] = jnp.ones((_L,), jnp.float32)

        pltpu.sync_copy(z_v, acc.at[pl.ds(s * _SPS, _SPS)])
        plsc.subcore_barrier()

        def chunk(m, _):
            r0 = (wid + m * _NW) * _MR
            pltpu.sync_copy(src_h.at[pl.ds(r0, _MR)], src_v)
            pltpu.sync_copy(dst_h.at[pl.ds(r0, _MR)], dst_v)
            pltpu.sync_copy(et_h.at[pl.ds(r0, _MR)], et_v)

            def vec(k, _):
                def vec16(j, _):
                    sl = pl.ds(j * _L, _L)
                    tv = et_v[k, sl]
                    gi_v[k, sl] = tv * _N + src_v[k, sl]
                    sg_v[k, sl] = dst_v[k, sl] * _R + tv
                    return 0
                lax.fori_loop(0, _CB // _L, vec16, 0)
                return 0
            lax.fori_loop(0, _MR, vec, 0)

            pltpu.sync_copy(gi_v, gidx_h.at[pl.ds(r0, _MR)])
            pltpu.sync_copy(sg_v, seg_h.at[pl.ds(r0, _MR)])
            cps = [pltpu.async_copy(ones_v, acc.at[sg_v.at[k]], sem, add=True)
                   for k in range(_MR)]
            for cp in cps:
                cp.wait()
            return 0
        lax.fori_loop(0, nm, chunk, 0)

        plsc.subcore_barrier()
        pltpu.sync_copy(acc.at[pl.ds(s * _SPS, _SPS)],
                        cnt_h.at[c, pl.ds(s * _SPS, _SPS)])

    return body(src2, dst2, et2)


def _sc_weights(winv, seg2):
    @functools.partial(
        pl.kernel,
        out_type=jax.ShapeDtypeStruct((_ROWS, _CB), jnp.float32),
        mesh=_mesh(),
        scratch_types=[
            pltpu.VMEM((_MR, _CB), jnp.int32),
            pltpu.VMEM((_MR, _CB), jnp.float32),
            pltpu.SemaphoreType.DMA,
        ],
    )
    def body(winv_h, seg_h, w_h, sg_v, w_v, sem):
        c = lax.axis_index("c")
        s = lax.axis_index("s")
        wid = s * _NC + c
        nm = (_NMAC - wid + _NW - 1) // _NW

        def chunk(m, _):
            r0 = (wid + m * _NW) * _MR
            pltpu.sync_copy(seg_h.at[pl.ds(r0, _MR)], sg_v)
            cps = [pltpu.async_copy(winv_h.at[sg_v.at[k]], w_v.at[k], sem)
                   for k in range(_MR)]
            for cp in cps:
                cp.wait()
            pltpu.sync_copy(w_v, w_h.at[pl.ds(r0, _MR)])
            return 0
        lax.fori_loop(0, nm, chunk, 0)

    return body(winv, seg2)


def _sc_edge(table, gidx2, w2, dst2, O):
    @functools.partial(
        pl.kernel,
        out_type=jax.ShapeDtypeStruct((_NC, _NP, O), jnp.float32),
        mesh=_mesh(),
        scratch_types=[
            pltpu.VMEM((_MR, _CB), jnp.int32),
            pltpu.VMEM((_MR, _CB), jnp.int32),
            pltpu.VMEM((_MR, _CB), jnp.float32),
            pltpu.VMEM((_WCB, O), jnp.float32),
            pltpu.VMEM((_ZR, O), jnp.float32),
            pltpu.VMEM_SHARED((_NP, O), jnp.float32),
            pltpu.SemaphoreType.DMA,
            pltpu.SemaphoreType.DMA,
            pltpu.SemaphoreType.DMA,
            pltpu.SemaphoreType.DMA,
            pltpu.SemaphoreType.DMA,
            pltpu.SemaphoreType.DMA,
        ],
    )
    def body(table_h, gidx_h, w_h, dst_h, parts_h,
             gi_v, d_v, w_v, rows_v, z_v, acc, g0, g1, g2, g3, s0, s1):
        c = lax.axis_index("c")
        s = lax.axis_index("s")
        wid = s * _NC + c
        nm = (_NMAC - wid + _NW - 1) // _NW

        def zrow(i, _):
            for k in range(O // _L):
                z_v[i, pl.ds(k * _L, _L)] = jnp.zeros((_L,), jnp.float32)
            return 0
        lax.fori_loop(0, _ZR, zrow, 0)

        def zcopy(t, _):
            pltpu.sync_copy(z_v, acc.at[pl.ds(s * _RPS + t * _ZR, _ZR)])
            return 0
        lax.fori_loop(0, _RPS // _ZR, zcopy, 0)
        plsc.subcore_barrier()

        def chunk(m, _):
            r0 = (wid + m * _NW) * _MR
            pltpu.sync_copy(gidx_h.at[pl.ds(r0, _MR)], gi_v)
            pltpu.sync_copy(w_h.at[pl.ds(r0, _MR)], w_v)
            pltpu.sync_copy(dst_h.at[pl.ds(r0, _MR)], d_v)

            gsem = [g0, g1, g2, g3]
            ssem = [s0, s1]
            gcps = {}
            scps = {}
            for k in range(2):
                gcps[k] = pltpu.async_copy(
                    table_h.at[gi_v.at[k]],
                    rows_v.at[pl.ds((k % _WM) * _CB, _CB)], gsem[k % _WM])
            for k in range(_MR):
                gcps[k].wait()

                def scale(q, _, k=k):
                    w16 = w_v[k, pl.ds(q * _L, _L)]
                    base = (k % _WM) * _CB
                    for ii in range(_L):
                        e = base + q * _L + ii
                        wb = jnp.full((_L,), w16[ii])
                        for kk in range(O // _L):
                            sl = pl.ds(kk * _L, _L)
                            rows_v[e, sl] = rows_v[e, sl] * wb
                    return 0
                lax.fori_loop(0, _CB // _L, scale, 0)

                scps[k] = pltpu.async_copy(
                    rows_v.at[pl.ds((k % _WM) * _CB, _CB)],
                    acc.at[d_v.at[k]], ssem[k % 2], add=True)
                scps[k].wait()
                if k + 2 < _MR:
                    gcps[k + 2] = pltpu.async_copy(
                        table_h.at[gi_v.at[k + 2]],
                        rows_v.at[pl.ds(((k + 2) % _WM) * _CB, _CB)],
                        gsem[(k + 2) % _WM])
            return 0
        lax.fori_loop(0, nm, chunk, 0)

        plsc.subcore_barrier()
        pltpu.sync_copy(acc.at[pl.ds(s * _RPS, _RPS)],
                        parts_h.at[c, pl.ds(s * _RPS, _RPS)])

    return body(table, gidx2, w2, dst2)


def _sc_edge3(table, src2, et2, w2, dst2):
    @functools.partial(
        pl.kernel,
        out_type=jax.ShapeDtypeStruct((_NC, _NP, _R * _C), jnp.float32),
        mesh=_mesh(),
        scratch_types=[
            pltpu.VMEM((_MR, _CB), jnp.int32),
            pltpu.VMEM((_MR, _CB), jnp.int32),
            pltpu.VMEM((_MR, _CB), jnp.int32),
            pltpu.VMEM((_MR, _CB), jnp.float32),
            pltpu.VMEM((_WCB, _R * _C), jnp.float32),
            pltpu.VMEM((_ZR, _R * _C), jnp.float32),
            pltpu.VMEM_SHARED((_NP, _R * _C), jnp.float32),
            pltpu.SemaphoreType.DMA,
            pltpu.SemaphoreType.DMA,
            pltpu.SemaphoreType.DMA,
            pltpu.SemaphoreType.DMA,
            pltpu.SemaphoreType.DMA,
            pltpu.SemaphoreType.DMA,
        ],
    )
    def body(table_h, src_h, et_h, w_h, dst_h, parts_h,
             s_v, t_v, d_v, w_v, rows_v, z_v, acc, g0, g1, g2, g3, s0, s1):
        c = lax.axis_index("c")
        s = lax.axis_index("s")
        wid = s * _NC + c
        nm = (_NMAC - wid + _NW - 1) // _NW

        def zrow(i, _):
            for k in range(_R * _C // _L):
                z_v[i, pl.ds(k * _L, _L)] = jnp.zeros((_L,), jnp.float32)
            return 0
        lax.fori_loop(0, _ZR, zrow, 0)

        def zcopy(t, _):
            pltpu.sync_copy(z_v, acc.at[pl.ds(s * _RPS + t * _ZR, _ZR)])
            return 0
        lax.fori_loop(0, _RPS // _ZR, zcopy, 0)
        plsc.subcore_barrier()

        def chunk(m, _):
            r0 = (wid + m * _NW) * _MR
            pltpu.sync_copy(src_h.at[pl.ds(r0, _MR)], s_v)
            pltpu.sync_copy(et_h.at[pl.ds(r0, _MR)], t_v)
            pltpu.sync_copy(dst_h.at[pl.ds(r0, _MR)], d_v)
            pltpu.sync_copy(w_h.at[pl.ds(r0, _MR)], w_v)

            gsem = [g0, g1, g2, g3]
            ssem = [s0, s1]
            gcps = {}
            scps = {}
            for k in range(2):
                gcps[k] = pltpu.async_copy(
                    table_h.at[s_v.at[k]],
                    rows_v.at[pl.ds((k % _WM) * _CB, _CB)], gsem[k % _WM])
            for k in range(_MR):
                gcps[k].wait()

                def scale(q, _, k=k):
                    w16 = w_v[k, pl.ds(q * _L, _L)]
                    t16 = t_v[k, pl.ds(q * _L, _L)]
                    base = (k % _WM) * _CB
                    for ii in range(_L):
                        e = base + q * _L + ii
                        tsc = t16[ii]
                        wsc = w16[ii]
                        for kk in range(_R):
                            fk = jnp.full((_L,), jnp.where(tsc == kk, wsc, 0.0))
                            sl = pl.ds(kk * _C, _C)
                            rows_v[e, sl] = rows_v[e, sl] * fk
                    return 0
                lax.fori_loop(0, _CB // _L, scale, 0)

                scps[k] = pltpu.async_copy(
                    rows_v.at[pl.ds((k % _WM) * _CB, _CB)],
                    acc.at[d_v.at[k]], ssem[k % 2], add=True)
                scps[k].wait()
                if k + 2 < _MR:
                    gcps[k + 2] = pltpu.async_copy(
                        table_h.at[s_v.at[k + 2]],
                        rows_v.at[pl.ds(((k + 2) % _WM) * _CB, _CB)],
                        gsem[(k + 2) % _WM])
            return 0
        lax.fori_loop(0, nm, chunk, 0)

        plsc.subcore_barrier()
        pltpu.sync_copy(acc.at[pl.ds(s * _RPS, _RPS)],
                        parts_h.at[c, pl.ds(s * _RPS, _RPS)])

    return body(table, src2, et2, w2, dst2)


def _tc_entry(x, num_x, W_num, b_num, a_in, W1):
    bn = 2000

    def body(x_r, nx_r, wn_r, b_r, a_r, w_r, h_r, xt_r):
        v = nx_r[...] * wn_r[...] + b_r[...]
        h = jnp.where(v >= 0, v, a_r[...] * v) + x_r[...]
        h_r[...] = h
        for r in range(_R):
            xt_r[r] = jax.lax.dot(h, w_r[r], precision=_HI)

    return pl.pallas_call(
        body,
        grid=(_N // bn,),
        in_specs=[
            pl.BlockSpec((bn, _D), lambda t: (t, 0)),
            pl.BlockSpec((bn, 1), lambda t: (t, 0)),
            pl.BlockSpec((1, _D), lambda t: (0, 0)),
            pl.BlockSpec((1, _D), lambda t: (0, 0)),
            pl.BlockSpec((1, _D), lambda t: (0, 0)),
            pl.BlockSpec((_R, _D, _H), lambda t: (0, 0, 0)),
        ],
        out_specs=[
            pl.BlockSpec((bn, _D), lambda t: (t, 0)),
            pl.BlockSpec((_R, bn, _H), lambda t: (0, t, 0)),
        ],
        out_shape=(
            jax.ShapeDtypeStruct((_N, _D), jnp.float32),
            jax.ShapeDtypeStruct((_R, _N, _H), jnp.float32),
        ),
    )(x, num_x, W_num, b_num.reshape(1, -1), a_in.reshape(1, -1), W1)


def _tc_winv(cnt_parts):
    def body(c_r, o_r):
        tot = c_r[0] + c_r[1]
        o_r[...] = 1.0 / jnp.maximum(tot, 1.0)

    out = pl.pallas_call(
        body,
        out_shape=jax.ShapeDtypeStruct((_SEGP // 128, 128), jnp.float32),
    )(cnt_parts.reshape(_NC, _SEGP // 128, 128))
    return out.reshape(_SEGP)


def _tc_wmats(c1, b1, c2, b2, c3, b3):
    def body(c1_r, b1_r, c2_r, b2_r, c3_r, b3_r, w1_r, w2_r, w3_r):
        w1_r[...] = jax.lax.dot(c1_r[...], b1_r[...], precision=_HI)
        w2_r[...] = jax.lax.dot(c2_r[...], b2_r[...], precision=_HI)
        w3_r[...] = jax.lax.dot(c3_r[...], b3_r[...], precision=_HI)

    return pl.pallas_call(
        body,
        out_shape=(
            jax.ShapeDtypeStruct((_R, _D * _H), jnp.float32),
            jax.ShapeDtypeStruct((_R, _H * _H), jnp.float32),
            jax.ShapeDtypeStruct((_R, _H * _C), jnp.float32),
        ),
    )(c1, b1.reshape(_NB, -1), c2, b2.reshape(_NB, -1), c3, b3.reshape(_NB, -1))


def _tc_layer(parts, h, root, bias, a, W):
    bn = 2000

    def body(p_r, h_r, r_r, b_r, a_r, w_r, o_r, xt_r):
        z = (p_r[0] + p_r[1] + b_r[...]
             + jax.lax.dot(h_r[...], r_r[...], precision=_HI))
        hn = jnp.where(z >= 0, z, a_r[...] * z)
        o_r[...] = hn
        for r in range(_R):
            xt_r[r] = jax.lax.dot(hn, w_r[r], precision=_HI)

    return pl.pallas_call(
        body,
        grid=(_N // bn,),
        in_specs=[
            pl.BlockSpec((_NC, bn, _H), lambda t: (0, t, 0)),
            pl.BlockSpec((bn, _D), lambda t: (t, 0)),
            pl.BlockSpec((_D, _H), lambda t: (0, 0)),
            pl.BlockSpec((1, _H), lambda t: (0, 0)),
            pl.BlockSpec((1, _H), lambda t: (0, 0)),
            pl.BlockSpec((_R, _H, _H), lambda t: (0, 0, 0)),
        ],
        out_specs=[
            pl.BlockSpec((bn, _H), lambda t: (t, 0)),
            pl.BlockSpec((_R, bn, _H), lambda t: (0, t, 0)),
        ],
        out_shape=(
            jax.ShapeDtypeStruct((_N, _H), jnp.float32),
            jax.ShapeDtypeStruct((_R, _N, _H), jnp.float32),
        ),
    )(parts, h, root, bias.reshape(1, -1), a.reshape(1, -1), W)


def _tc_layer3(parts, h, root, bias, a, Wc):
    bn = 2000

    def body(p_r, h_r, r_r, b_r, a_r, w_r, o_r, xt_r):
        z = (p_r[0] + p_r[1] + b_r[...]
             + jax.lax.dot(h_r[...], r_r[...], precision=_HI))
        hn = jnp.where(z >= 0, z, a_r[...] * z)
        o_r[...] = hn
        xt_r[...] = jax.lax.dot(hn, w_r[...], precision=_HI)

    return pl.pallas_call(
        body,
        grid=(_N // bn,),
        in_specs=[
            pl.BlockSpec((_NC, bn, _H), lambda t: (0, t, 0)),
            pl.BlockSpec((bn, _D), lambda t: (t, 0)),
            pl.BlockSpec((_D, _H), lambda t: (0, 0)),
            pl.BlockSpec((1, _H), lambda t: (0, 0)),
            pl.BlockSpec((1, _H), lambda t: (0, 0)),
            pl.BlockSpec((_D, _R * _C), lambda t: (0, 0)),
        ],
        out_specs=[
            pl.BlockSpec((bn, _H), lambda t: (t, 0)),
            pl.BlockSpec((bn, _R * _C), lambda t: (t, 0)),
        ],
        out_shape=(
            jax.ShapeDtypeStruct((_N, _H), jnp.float32),
            jax.ShapeDtypeStruct((_N, _R * _C), jnp.float32),
        ),
    )(parts, h, root, bias.reshape(1, -1), a.reshape(1, -1), Wc)


def _tc_final(parts, h, root, bias):
    bn = 2000

    def body(p_r, h_r, r_r, b_r, o_r):
        p = p_r[0] + p_r[1]
        agg = p[:, 0:_C]
        for k in range(1, _R):
            agg = agg + p[:, k * _C:(k + 1) * _C]
        z = (agg + b_r[...]
             + jax.lax.dot(h_r[...], r_r[...], precision=_HI))
        m = jnp.max(z, axis=1, keepdims=True)
        zs = z - m
        o_r[...] = zs - jnp.log(jnp.sum(jnp.exp(zs), axis=1, keepdims=True))

    return pl.pallas_call(
        body,
        grid=(_N // bn,),
        in_specs=[
            pl.BlockSpec((_NC, bn, _R * _C), lambda t: (0, t, 0)),
            pl.BlockSpec((bn, _D), lambda t: (t, 0)),
            pl.BlockSpec((_D, _C), lambda t: (0, 0)),
            pl.BlockSpec((1, _C), lambda t: (0, 0)),
        ],
        out_specs=pl.BlockSpec((bn, _C), lambda t: (t, 0)),
        out_shape=jax.ShapeDtypeStruct((_N, _C), jnp.float32),
    )(parts, h, root, bias.reshape(1, -1))


def kernel(x, num_x, W_num, b_num, a_in,
           comp1, bases1, root1, bias1, a1,
           comp2, bases2, root2, bias2, a2,
           comp3, bases3, root3, bias3,
           edge_index, edge_type):
    src2 = edge_index[0].reshape(_ROWS, _CB)
    dst2 = edge_index[1].reshape(_ROWS, _CB)
    et2 = edge_type.reshape(_ROWS, _CB)

    cnt_parts, gidx2, seg2 = _sc_prep(src2, dst2, et2)
    winv = _tc_winv(cnt_parts)
    w2 = _sc_weights(winv, seg2)

    w1f, w2f, w3f = _tc_wmats(comp1, bases1, comp2, bases2, comp3, bases3)
    W1 = w1f.reshape(_R, _D, _H)
    W2 = w2f.reshape(_R, _H, _H)
    W3c = w3f.reshape(_R, _H, _C).transpose(1, 0, 2).reshape(_H, _R * _C)

    h0, xt1 = _tc_entry(x, num_x, W_num, b_num, a_in, W1)
    parts1 = _sc_edge(xt1.reshape(_R * _N, _H), gidx2, w2, dst2, _H)

    h1, xt2 = _tc_layer(parts1, h0, root1, bias1, a1, W2)
    parts2 = _sc_edge(xt2.reshape(_R * _N, _H), gidx2, w2, dst2, _H)

    h2, xt3 = _tc_layer3(parts2, h1, root2, bias2, a2, W3c)
    parts3 = _sc_edge3(xt3, src2, et2, w2, dst2)
    return _tc_final(parts3, h2, root3, bias3)


# speedup vs baseline: 1.0781x; 1.0781x over previous
import functools

import jax
import jax.numpy as jnp
from jax import lax
from jax.experimental import pallas as pl
from jax.experimental.pallas import tpu as pltpu
from jax.experimental.pallas import tpu_sc as plsc

_N = 10000
_E = 320000
_D = 128
_H = 128
_R = 8
_NB = 8
_C = 16

_NC, _NS, _L = 2, 16, 16
_NW = _NC * _NS
_EPW = _E // _NW
_CB = 64
_MR = 8
_WM = 4
_WCB = _CB * _WM
_MCB = _CB * _MR
_ROWS = _E // _CB
_NMAC = _ROWS // _MR
_BPR = _CB // _L
_SEGP = 81920
_SPS = _SEGP // _NS
_NP = 10240
_RPS = _NP // _NS
_ZR = 8

_HI = lax.Precision.HIGHEST


def _mesh():
    return plsc.VectorSubcoreMesh(
        core_axis_name="c", subcore_axis_name="s",
        num_cores=_NC, num_subcores=_NS)


def _sc_prep(src2, dst2, et2):
    @functools.partial(
        pl.kernel,
        out_type=(
            jax.ShapeDtypeStruct((_NC, _SEGP), jnp.float32),
            jax.ShapeDtypeStruct((_ROWS, _CB), jnp.int32),
            jax.ShapeDtypeStruct((_ROWS, _CB), jnp.int32),
        ),
        mesh=_mesh(),
        scratch_types=[
            pltpu.VMEM((_MR, _CB), jnp.int32),
            pltpu.VMEM((_MR, _CB), jnp.int32),
            pltpu.VMEM((_MR, _CB), jnp.int32),
            pltpu.VMEM((_MR, _CB), jnp.int32),
            pltpu.VMEM((_MR, _CB), jnp.int32),
            pltpu.VMEM((_CB,), jnp.float32),
            pltpu.VMEM((_SPS,), jnp.float32),
            pltpu.VMEM_SHARED((_SEGP,), jnp.float32),
            pltpu.SemaphoreType.DMA,
        ],
    )
    def body(src_h, dst_h, et_h, cnt_h, gidx_h, seg_h,
             src_v, dst_v, et_v, gi_v, sg_v, ones_v, z_v, acc, sem):
        c = lax.axis_index("c")
        s = lax.axis_index("s")
        wid = s * _NC + c
        nm = (_NMAC - wid + _NW - 1) // _NW

        def initz(i, _):
            z_v[pl.ds(i * _L, _L)] = jnp.zeros((_L,), jnp.float32)
            return 0
        lax.fori_loop(0, _SPS // _L, initz, 0)

        for j in range(_CB // _L):
            ones_v[pl.ds(j * _L, _L)] = jnp.ones((_L,), jnp.float32)

        pltpu.sync_copy(z_v, acc.at[pl.ds(s * _SPS, _SPS)])
        plsc.subcore_barrier()

        def chunk(m, _):
            r0 = (wid + m * _NW) * _MR
            pltpu.sync_copy(src_h.at[pl.ds(r0, _MR)], src_v)
            pltpu.sync_copy(dst_h.at[pl.ds(r0, _MR)], dst_v)
            pltpu.sync_copy(et_h.at[pl.ds(r0, _MR)], et_v)

            def vec(k, _):
                def vec16(j, _):
                    sl = pl.ds(j * _L, _L)
                    tv = et_v[k, sl]
                    gi_v[k, sl] = tv * _N + src_v[k, sl]
                    sg_v[k, sl] = dst_v[k, sl] * _R + tv
                    return 0
                lax.fori_loop(0, _CB // _L, vec16, 0)
                return 0
            lax.fori_loop(0, _MR, vec, 0)

            pltpu.sync_copy(gi_v, gidx_h.at[pl.ds(r0, _MR)])
            pltpu.sync_copy(sg_v, seg_h.at[pl.ds(r0, _MR)])
            cps = [pltpu.async_copy(ones_v, acc.at[sg_v.at[k]], sem, add=True)
                   for k in range(_MR)]
            for cp in cps:
                cp.wait()
            return 0
        lax.fori_loop(0, nm, chunk, 0)

        plsc.subcore_barrier()
        pltpu.sync_copy(acc.at[pl.ds(s * _SPS, _SPS)],
                        cnt_h.at[c, pl.ds(s * _SPS, _SPS)])

    return body(src2, dst2, et2)


def _sc_weights(winv, seg2):
    @functools.partial(
        pl.kernel,
        out_type=jax.ShapeDtypeStruct((_ROWS, _CB), jnp.float32),
        mesh=_mesh(),
        scratch_types=[
            pltpu.VMEM((_MR, _CB), jnp.int32),
            pltpu.VMEM((_MR, _CB), jnp.float32),
            pltpu.SemaphoreType.DMA,
        ],
    )
    def body(winv_h, seg_h, w_h, sg_v, w_v, sem):
        c = lax.axis_index("c")
        s = lax.axis_index("s")
        wid = s * _NC + c
        nm = (_NMAC - wid + _NW - 1) // _NW

        def chunk(m, _):
            r0 = (wid + m * _NW) * _MR
            pltpu.sync_copy(seg_h.at[pl.ds(r0, _MR)], sg_v)
            cps = [pltpu.async_copy(winv_h.at[sg_v.at[k]], w_v.at[k], sem)
                   for k in range(_MR)]
            for cp in cps:
                cp.wait()
            pltpu.sync_copy(w_v, w_h.at[pl.ds(r0, _MR)])
            return 0
        lax.fori_loop(0, nm, chunk, 0)

    return body(winv, seg2)


def _sc_edge(table, gidx2, w2, dst2, O):
    @functools.partial(
        pl.kernel,
        out_type=jax.ShapeDtypeStruct((_NC, _NP, O), jnp.float32),
        mesh=_mesh(),
        scratch_types=[
            pltpu.VMEM((_MR, _CB), jnp.int32),
            pltpu.VMEM((_MR, _CB), jnp.int32),
            pltpu.VMEM((_MR, _CB), jnp.float32),
            pltpu.VMEM((_WCB, O), jnp.float32),
            pltpu.VMEM((_ZR, O), jnp.float32),
            pltpu.VMEM_SHARED((_NP, O), jnp.float32),
            pltpu.SemaphoreType.DMA,
            pltpu.SemaphoreType.DMA,
            pltpu.SemaphoreType.DMA,
            pltpu.SemaphoreType.DMA,
            pltpu.SemaphoreType.DMA,
            pltpu.SemaphoreType.DMA,
        ],
    )
    def body(table_h, gidx_h, w_h, dst_h, parts_h,
             gi_v, d_v, w_v, rows_v, z_v, acc, g0, g1, g2, g3, s0, s1):
        c = lax.axis_index("c")
        s = lax.axis_index("s")
        wid = s * _NC + c
        nm = (_NMAC - wid + _NW - 1) // _NW

        def zrow(i, _):
            for k in range(O // _L):
                z_v[i, pl.ds(k * _L, _L)] = jnp.zeros((_L,), jnp.float32)
            return 0
        lax.fori_loop(0, _ZR, zrow, 0)

        def zcopy(t, _):
            pltpu.sync_copy(z_v, acc.at[pl.ds(s * _RPS + t * _ZR, _ZR)])
            return 0
        lax.fori_loop(0, _RPS // _ZR, zcopy, 0)
        plsc.subcore_barrier()

        def chunk(m, _):
            r0 = (wid + m * _NW) * _MR
            pltpu.sync_copy(gidx_h.at[pl.ds(r0, _MR)], gi_v)
            pltpu.sync_copy(w_h.at[pl.ds(r0, _MR)], w_v)
            pltpu.sync_copy(dst_h.at[pl.ds(r0, _MR)], d_v)

            gsem = [g0, g1, g2, g3]
            ssem = [s0, s1]
            gcps = {}
            scps = {}
            for k in range(2):
                gcps[k] = pltpu.async_copy(
                    table_h.at[gi_v.at[k]],
                    rows_v.at[pl.ds((k % _WM) * _CB, _CB)], gsem[k % _WM])
            for k in range(_MR):
                gcps[k].wait()

                def scale(q, _, k=k):
                    w16 = w_v[k, pl.ds(q * _L, _L)]
                    base = (k % _WM) * _CB
                    for ii in range(_L):
                        e = base + q * _L + ii
                        wb = jnp.full((_L,), w16[ii])
                        for kk in range(O // _L):
                            sl = pl.ds(kk * _L, _L)
                            rows_v[e, sl] = rows_v[e, sl] * wb
                    return 0
                lax.fori_loop(0, _CB // _L, scale, 0)

                if k >= 1:
                    scps[k - 1].wait()
                scps[k] = pltpu.async_copy(
                    rows_v.at[pl.ds((k % _WM) * _CB, _CB)],
                    acc.at[d_v.at[k]], ssem[k % 2], add=True)
                if k + 2 < _MR:
                    gcps[k + 2] = pltpu.async_copy(
                        table_h.at[gi_v.at[k + 2]],
                        rows_v.at[pl.ds(((k + 2) % _WM) * _CB, _CB)],
                        gsem[(k + 2) % _WM])
            scps[_MR - 1].wait()
            return 0
        lax.fori_loop(0, nm, chunk, 0)

        plsc.subcore_barrier()
        pltpu.sync_copy(acc.at[pl.ds(s * _RPS, _RPS)],
                        parts_h.at[c, pl.ds(s * _RPS, _RPS)])

    return body(table, gidx2, w2, dst2)


def _sc_edge3(table, src2, et2, w2, dst2):
    @functools.partial(
        pl.kernel,
        out_type=jax.ShapeDtypeStruct((_NC, _NP, _R * _C), jnp.float32),
        mesh=_mesh(),
        scratch_types=[
            pltpu.VMEM((_MR, _CB), jnp.int32),
            pltpu.VMEM((_MR, _CB), jnp.int32),
            pltpu.VMEM((_MR, _CB), jnp.int32),
            pltpu.VMEM((_MR, _CB), jnp.float32),
            pltpu.VMEM((_WCB, _R * _C), jnp.float32),
            pltpu.VMEM((_ZR, _R * _C), jnp.float32),
            pltpu.VMEM_SHARED((_NP, _R * _C), jnp.float32),
            pltpu.SemaphoreType.DMA,
            pltpu.SemaphoreType.DMA,
            pltpu.SemaphoreType.DMA,
            pltpu.SemaphoreType.DMA,
            pltpu.SemaphoreType.DMA,
            pltpu.SemaphoreType.DMA,
        ],
    )
    def body(table_h, src_h, et_h, w_h, dst_h, parts_h,
             s_v, t_v, d_v, w_v, rows_v, z_v, acc, g0, g1, g2, g3, s0, s1):
        c = lax.axis_index("c")
        s = lax.axis_index("s")
        wid = s * _NC + c
        nm = (_NMAC - wid + _NW - 1) // _NW

        def zrow(i, _):
            for k in range(_R * _C // _L):
                z_v[i, pl.ds(k * _L, _L)] = jnp.zeros((_L,), jnp.float32)
            return 0
        lax.fori_loop(0, _ZR, zrow, 0)

        def zcopy(t, _):
            pltpu.sync_copy(z_v, acc.at[pl.ds(s * _RPS + t * _ZR, _ZR)])
            return 0
        lax.fori_loop(0, _RPS // _ZR, zcopy, 0)
        plsc.subcore_barrier()

        def chunk(m, _):
            r0 = (wid + m * _NW) * _MR
            pltpu.sync_copy(src_h.at[pl.ds(r0, _MR)], s_v)
            pltpu.sync_copy(et_h.at[pl.ds(r0, _MR)], t_v)
            pltpu.sync_copy(dst_h.at[pl.ds(r0, _MR)], d_v)
            pltpu.sync_copy(w_h.at[pl.ds(r0, _MR)], w_v)

            gsem = [g0, g1, g2, g3]
            ssem = [s0, s1]
            gcps = {}
            scps = {}
            for k in range(2):
                gcps[k] = pltpu.async_copy(
                    table_h.at[s_v.at[k]],
                    rows_v.at[pl.ds((k % _WM) * _CB, _CB)], gsem[k % _WM])
            for k in range(_MR):
                gcps[k].wait()

                def scale(q, _, k=k):
                    w16 = w_v[k, pl.ds(q * _L, _L)]
                    t16 = t_v[k, pl.ds(q * _L, _L)]
                    base = (k % _WM) * _CB
                    for ii in range(_L):
                        e = base + q * _L + ii
                        tsc = t16[ii]
                        wsc = w16[ii]
                        for kk in range(_R):
                            fk = jnp.full((_L,), jnp.where(tsc == kk, wsc, 0.0))
                            sl = pl.ds(kk * _C, _C)
                            rows_v[e, sl] = rows_v[e, sl] * fk
                    return 0
                lax.fori_loop(0, _CB // _L, scale, 0)

                if k >= 1:
                    scps[k - 1].wait()
                scps[k] = pltpu.async_copy(
                    rows_v.at[pl.ds((k % _WM) * _CB, _CB)],
                    acc.at[d_v.at[k]], ssem[k % 2], add=True)
                if k + 2 < _MR:
                    gcps[k + 2] = pltpu.async_copy(
                        table_h.at[s_v.at[k + 2]],
                        rows_v.at[pl.ds(((k + 2) % _WM) * _CB, _CB)],
                        gsem[(k + 2) % _WM])
            scps[_MR - 1].wait()
            return 0
        lax.fori_loop(0, nm, chunk, 0)

        plsc.subcore_barrier()
        pltpu.sync_copy(acc.at[pl.ds(s * _RPS, _RPS)],
                        parts_h.at[c, pl.ds(s * _RPS, _RPS)])

    return body(table, src2, et2, w2, dst2)


def _tc_entry(x, num_x, W_num, b_num, a_in, W1):
    bn = 2000

    def body(x_r, nx_r, wn_r, b_r, a_r, w_r, h_r, xt_r):
        v = nx_r[...] * wn_r[...] + b_r[...]
        h = jnp.where(v >= 0, v, a_r[...] * v) + x_r[...]
        h_r[...] = h
        for r in range(_R):
            xt_r[r] = jax.lax.dot(h, w_r[r], precision=_HI)

    return pl.pallas_call(
        body,
        grid=(_N // bn,),
        in_specs=[
            pl.BlockSpec((bn, _D), lambda t: (t, 0)),
            pl.BlockSpec((bn, 1), lambda t: (t, 0)),
            pl.BlockSpec((1, _D), lambda t: (0, 0)),
            pl.BlockSpec((1, _D), lambda t: (0, 0)),
            pl.BlockSpec((1, _D), lambda t: (0, 0)),
            pl.BlockSpec((_R, _D, _H), lambda t: (0, 0, 0)),
        ],
        out_specs=[
            pl.BlockSpec((bn, _D), lambda t: (t, 0)),
            pl.BlockSpec((_R, bn, _H), lambda t: (0, t, 0)),
        ],
        out_shape=(
            jax.ShapeDtypeStruct((_N, _D), jnp.float32),
            jax.ShapeDtypeStruct((_R, _N, _H), jnp.float32),
        ),
    )(x, num_x, W_num, b_num.reshape(1, -1), a_in.reshape(1, -1), W1)


def _tc_winv(cnt_parts):
    def body(c_r, o_r):
        tot = c_r[0] + c_r[1]
        o_r[...] = 1.0 / jnp.maximum(tot, 1.0)

    out = pl.pallas_call(
        body,
        out_shape=jax.ShapeDtypeStruct((_SEGP // 128, 128), jnp.float32),
    )(cnt_parts.reshape(_NC, _SEGP // 128, 128))
    return out.reshape(_SEGP)


def _tc_wmats(c1, b1, c2, b2, c3, b3):
    def body(c1_r, b1_r, c2_r, b2_r, c3_r, b3_r, w1_r, w2_r, w3_r):
        w1_r[...] = jax.lax.dot(c1_r[...], b1_r[...], precision=_HI)
        w2_r[...] = jax.lax.dot(c2_r[...], b2_r[...], precision=_HI)
        w3_r[...] = jax.lax.dot(c3_r[...], b3_r[...], precision=_HI)

    return pl.pallas_call(
        body,
        out_shape=(
            jax.ShapeDtypeStruct((_R, _D * _H), jnp.float32),
            jax.ShapeDtypeStruct((_R, _H * _H), jnp.float32),
            jax.ShapeDtypeStruct((_R, _H * _C), jnp.float32),
        ),
    )(c1, b1.reshape(_NB, -1), c2, b2.reshape(_NB, -1), c3, b3.reshape(_NB, -1))


def _tc_layer(parts, h, root, bias, a, W):
    bn = 2000

    def body(p_r, h_r, r_r, b_r, a_r, w_r, o_r, xt_r):
        z = (p_r[0] + p_r[1] + b_r[...]
             + jax.lax.dot(h_r[...], r_r[...], precision=_HI))
        hn = jnp.where(z >= 0, z, a_r[...] * z)
        o_r[...] = hn
        for r in range(_R):
            xt_r[r] = jax.lax.dot(hn, w_r[r], precision=_HI)

    return pl.pallas_call(
        body,
        grid=(_N // bn,),
        in_specs=[
            pl.BlockSpec((_NC, bn, _H), lambda t: (0, t, 0)),
            pl.BlockSpec((bn, _D), lambda t: (t, 0)),
            pl.BlockSpec((_D, _H), lambda t: (0, 0)),
            pl.BlockSpec((1, _H), lambda t: (0, 0)),
            pl.BlockSpec((1, _H), lambda t: (0, 0)),
            pl.BlockSpec((_R, _H, _H), lambda t: (0, 0, 0)),
        ],
        out_specs=[
            pl.BlockSpec((bn, _H), lambda t: (t, 0)),
            pl.BlockSpec((_R, bn, _H), lambda t: (0, t, 0)),
        ],
        out_shape=(
            jax.ShapeDtypeStruct((_N, _H), jnp.float32),
            jax.ShapeDtypeStruct((_R, _N, _H), jnp.float32),
        ),
    )(parts, h, root, bias.reshape(1, -1), a.reshape(1, -1), W)


def _tc_layer3(parts, h, root, bias, a, Wc):
    bn = 2000

    def body(p_r, h_r, r_r, b_r, a_r, w_r, o_r, xt_r):
        z = (p_r[0] + p_r[1] + b_r[...]
             + jax.lax.dot(h_r[...], r_r[...], precision=_HI))
        hn = jnp.where(z >= 0, z, a_r[...] * z)
        o_r[...] = hn
        xt_r[...] = jax.lax.dot(hn, w_r[...], precision=_HI)

    return pl.pallas_call(
        body,
        grid=(_N // bn,),
        in_specs=[
            pl.BlockSpec((_NC, bn, _H), lambda t: (0, t, 0)),
            pl.BlockSpec((bn, _D), lambda t: (t, 0)),
            pl.BlockSpec((_D, _H), lambda t: (0, 0)),
            pl.BlockSpec((1, _H), lambda t: (0, 0)),
            pl.BlockSpec((1, _H), lambda t: (0, 0)),
            pl.BlockSpec((_D, _R * _C), lambda t: (0, 0)),
        ],
        out_specs=[
            pl.BlockSpec((bn, _H), lambda t: (t, 0)),
            pl.BlockSpec((bn, _R * _C), lambda t: (t, 0)),
        ],
        out_shape=(
            jax.ShapeDtypeStruct((_N, _H), jnp.float32),
            jax.ShapeDtypeStruct((_N, _R * _C), jnp.float32),
        ),
    )(parts, h, root, bias.reshape(1, -1), a.reshape(1, -1), Wc)


def _tc_final(parts, h, root, bias):
    bn = 2000

    def body(p_r, h_r, r_r, b_r, o_r):
        p = p_r[0] + p_r[1]
        agg = p[:, 0:_C]
        for k in range(1, _R):
            agg = agg + p[:, k * _C:(k + 1) * _C]
        z = (agg + b_r[...]
             + jax.lax.dot(h_r[...], r_r[...], precision=_HI))
        m = jnp.max(z, axis=1, keepdims=True)
        zs = z - m
        o_r[...] = zs - jnp.log(jnp.sum(jnp.exp(zs), axis=1, keepdims=True))

    return pl.pallas_call(
        body,
        grid=(_N // bn,),
        in_specs=[
            pl.BlockSpec((_NC, bn, _R * _C), lambda t: (0, t, 0)),
            pl.BlockSpec((bn, _D), lambda t: (t, 0)),
            pl.BlockSpec((_D, _C), lambda t: (0, 0)),
            pl.BlockSpec((1, _C), lambda t: (0, 0)),
        ],
        out_specs=pl.BlockSpec((bn, _C), lambda t: (t, 0)),
        out_shape=jax.ShapeDtypeStruct((_N, _C), jnp.float32),
    )(parts, h, root, bias.reshape(1, -1))


def kernel(x, num_x, W_num, b_num, a_in,
           comp1, bases1, root1, bias1, a1,
           comp2, bases2, root2, bias2, a2,
           comp3, bases3, root3, bias3,
           edge_index, edge_type):
    src2 = edge_index[0].reshape(_ROWS, _CB)
    dst2 = edge_index[1].reshape(_ROWS, _CB)
    et2 = edge_type.reshape(_ROWS, _CB)

    cnt_parts, gidx2, seg2 = _sc_prep(src2, dst2, et2)
    winv = _tc_winv(cnt_parts)
    w2 = _sc_weights(winv, seg2)

    w1f, w2f, w3f = _tc_wmats(comp1, bases1, comp2, bases2, comp3, bases3)
    W1 = w1f.reshape(_R, _D, _H)
    W2 = w2f.reshape(_R, _H, _H)
    W3c = w3f.reshape(_R, _H, _C).transpose(1, 0, 2).reshape(_H, _R * _C)

    h0, xt1 = _tc_entry(x, num_x, W_num, b_num, a_in, W1)
    parts1 = _sc_edge(xt1.reshape(_R * _N, _H), gidx2, w2, dst2, _H)

    h1, xt2 = _tc_layer(parts1, h0, root1, bias1, a1, W2)
    parts2 = _sc_edge(xt2.reshape(_R * _N, _H), gidx2, w2, dst2, _H)

    h2, xt3 = _tc_layer3(parts2, h1, root2, bias2, a2, W3c)
    parts3 = _sc_edge3(xt3, src2, et2, w2, dst2)
    return _tc_final(parts3, h2, root3, bias3)
